# Initial kernel scaffold; baseline (speedup 1.0000x reference)
import jax
import jax.numpy as jnp
from jax.experimental import pallas as pl


def kernel(X, A, W0, b0, W1, b1, Wm, bm):
    raise NotImplementedError("write your pallas kernel here")


# speedup vs baseline: 8.8645x; 8.8645x over previous
import functools

import jax
import jax.numpy as jnp
from jax import lax
from jax.experimental import pallas as pl
from jax.experimental.pallas import tpu as pltpu
from jax.experimental.pallas import tpu_sc as plsc

N = 10000
D = 128
NCLS = 64
E = 320000
NC = 2
NS = 16
NW = NC * NS
CH = 128
RPW = 80
EP = NW * RPW * CH
ROWS = EP // CH
ACC_N = 10240
ZR = ACC_N // NS
R = 1000
G = N // R

_sc_mesh = plsc.VectorSubcoreMesh(core_axis_name="c", subcore_axis_name="s")


@functools.partial(
    pl.kernel,
    out_type=jax.ShapeDtypeStruct((NC, ACC_N, 16), jnp.float32),
    mesh=_sc_mesh,
    scratch_types=[
        pltpu.VMEM((RPW, CH), jnp.int32),
        pltpu.VMEM((CH, 16), jnp.float32),
        pltpu.VMEM_SHARED((ACC_N, 16), jnp.float32),
    ],
)
def _deg_count(dstr_hbm, ones_hbm, zeros_hbm, deg_hbm, dst_v, ones_v, acc):
    c = lax.axis_index("c")
    s = lax.axis_index("s")
    wid = c * NS + s
    pltpu.sync_copy(zeros_hbm, acc.at[pl.ds(s * ZR, ZR)])
    pltpu.sync_copy(dstr_hbm.at[pl.ds(wid * RPW, RPW)], dst_v)
    pltpu.sync_copy(ones_hbm, ones_v)
    plsc.subcore_barrier()

    def body(j, carry):
        pltpu.sync_copy(ones_v, acc.at[dst_v.at[j]], add=True)
        return carry

    lax.fori_loop(0, RPW, body, 0)
    plsc.subcore_barrier()
    pltpu.sync_copy(acc.at[pl.ds(s * ZR, ZR)], deg_hbm.at[c, pl.ds(s * ZR, ZR)])


@functools.partial(
    pl.kernel,
    out_type=jax.ShapeDtypeStruct((NC, ACC_N, D), jnp.float32),
    mesh=_sc_mesh,
    scratch_types=[
        pltpu.VMEM((RPW, CH), jnp.int32),
        pltpu.VMEM((RPW, CH), jnp.int32),
        pltpu.VMEM((CH, D), jnp.float32),
        pltpu.VMEM_SHARED((ACC_N, D), jnp.float32),
        pltpu.SemaphoreType.DMA,
    ],
)
def _edge_scatter(y_hbm, srcr_hbm, dstr_hbm, zeros_hbm, z_hbm,
                  src_v, dst_v, rows_v, acc, gsem):
    c = lax.axis_index("c")
    s = lax.axis_index("s")
    wid = c * NS + s
    pltpu.sync_copy(zeros_hbm, acc.at[pl.ds(s * ZR, ZR)])
    pltpu.sync_copy(srcr_hbm.at[pl.ds(wid * RPW, RPW)], src_v)
    pltpu.sync_copy(dstr_hbm.at[pl.ds(wid * RPW, RPW)], dst_v)
    plsc.subcore_barrier()

    def body(j, carry):
        pltpu.async_copy(y_hbm.at[src_v.at[j]], rows_v, gsem).wait()
        pltpu.sync_copy(rows_v, acc.at[dst_v.at[j]], add=True)
        return carry

    lax.fori_loop(0, RPW, body, 0)
    plsc.subcore_barrier()
    pltpu.sync_copy(acc.at[pl.ds(s * ZR, ZR)], z_hbm.at[c, pl.ds(s * ZR, ZR)])


def _dinv(degp_ref):
    deg = degp_ref[0, :, 0:1] + degp_ref[1, :, 0:1] + 1.0
    return lax.rsqrt(deg)


def _mm(x, w_ref):
    return lax.dot_general(x, w_ref[...], (((1,), (1,)), ((), ())),
                           preferred_element_type=jnp.float32)


def _tc1_body(x_ref, w0_ref, degp_ref, xw_ref, y_ref):
    dinv = _dinv(degp_ref)
    xw = _mm(x_ref[...], w0_ref)
    xw_ref[...] = xw
    y_ref[...] = xw * dinv


def _tc2_body(z_ref, xw0_ref, degp_ref, w1_ref, b0_ref, xw1_ref, y1_ref):
    dinv = _dinv(degp_ref)
    h = dinv * (z_ref[0] + z_ref[1]) + (dinv * dinv) * xw0_ref[...] + b0_ref[...]
    h = jnp.maximum(h, 0.0)
    xw1 = _mm(h, w1_ref)
    xw1_ref[...] = xw1
    y1_ref[...] = xw1 * dinv


def _tc3_body(z_ref, xw1_ref, degp_ref, wm_ref, b1_ref, bm_ref, out_ref):
    dinv = _dinv(degp_ref)
    h = dinv * (z_ref[0] + z_ref[1]) + (dinv * dinv) * xw1_ref[...] + b1_ref[...]
    out_ref[...] = _mm(h, wm_ref) + bm_ref[...]


_deg_spec = pl.BlockSpec((NC, R, 16), lambda i: (0, i, 0))
_row_spec = pl.BlockSpec((R, D), lambda i: (i, 0))
_z_spec = pl.BlockSpec((NC, R, D), lambda i: (0, i, 0))
_w_spec = pl.BlockSpec((D, D), lambda i: (0, 0))

_tc1 = pl.pallas_call(
    _tc1_body,
    grid=(G,),
    in_specs=[_row_spec, _w_spec, _deg_spec],
    out_specs=[_row_spec, _row_spec],
    out_shape=[jax.ShapeDtypeStruct((N, D), jnp.float32)] * 2,
)

_tc2 = pl.pallas_call(
    _tc2_body,
    grid=(G,),
    in_specs=[_z_spec, _row_spec, _deg_spec, _w_spec,
              pl.BlockSpec((1, D), lambda i: (0, 0))],
    out_specs=[_row_spec, _row_spec],
    out_shape=[jax.ShapeDtypeStruct((N, D), jnp.float32)] * 2,
)

_tc3 = pl.pallas_call(
    _tc3_body,
    grid=(G,),
    in_specs=[_z_spec, _row_spec, _deg_spec,
              pl.BlockSpec((NCLS, D), lambda i: (0, 0)),
              pl.BlockSpec((1, D), lambda i: (0, 0)),
              pl.BlockSpec((1, NCLS), lambda i: (0, 0))],
    out_specs=pl.BlockSpec((R, NCLS), lambda i: (i, 0)),
    out_shape=jax.ShapeDtypeStruct((N, NCLS), jnp.float32),
)


def kernel(X, A, W0, b0, W1, b1, Wm, bm):
    pad = EP - E
    srcr = jnp.concatenate(
        [A[0], jnp.zeros((pad,), jnp.int32)]).reshape(ROWS, CH)
    dstr = jnp.concatenate(
        [A[1], jnp.full((pad,), N, jnp.int32)]).reshape(ROWS, CH)
    zeros = jnp.zeros((ZR, D), jnp.float32)
    zeros16 = jnp.zeros((ZR, 16), jnp.float32)
    ones16 = jnp.ones((CH, 16), jnp.float32)

    degp = _deg_count(dstr, ones16, zeros16)
    xw0, y0 = _tc1(X, W0, degp)
    z0 = _edge_scatter(y0, srcr, dstr, zeros)
    xw1, y1 = _tc2(z0, xw0, degp, W1, b0.reshape(1, D))
    z1 = _edge_scatter(y1, srcr, dstr, zeros)
    return _tc3(z1, xw1, degp, Wm, b1.reshape(1, D), bm.reshape(1, NCLS))
